# baseline (speedup 1.0000x reference)
import functools

import jax
import jax.numpy as jnp
from jax import lax
from jax.experimental import pallas as pl
from jax.experimental.pallas import tpu as pltpu
from jax.experimental.pallas import tpu_sc as plsc

VOCAB = 1000000
EMB = 64
BATCH = 16384
NEG = 20
NP1 = NEG + 1

NC, NS = 2, 16
NW = NC * NS

BW = BATCH // NW
GB = 16
NBLK = BW // GB
CH = 128
TROWS = BW // CH
NROWS = BW * NEG // CH
NCHB = GB * NEG // 64
PN = 24

DCOLS = 128
DROWS = BATCH * NP1 // DCOLS
DW = DROWS // NW


def _dots_body(emb_in, emb_out, idx_t, idx_c, idx_n, par_t, par_c, par_n,
               dots_out,
               itv, icv, inv, ptv, pcv, pnv, vbuf, ubuf, nbuf, dots,
               sem0, sem1):
    c = lax.axis_index("c")
    s = lax.axis_index("s")
    wid = s * NC + c

    pltpu.sync_copy(idx_t.at[pl.ds(wid * BW, BW)], itv)
    pltpu.sync_copy(idx_c.at[pl.ds(wid * BW, BW)], icv)
    pltpu.sync_copy(idx_n.at[pl.ds(wid * BW * NEG, BW * NEG)], inv)
    pltpu.sync_copy(par_t.at[pl.ds(wid * BW, BW)], ptv)
    pltpu.sync_copy(par_c.at[pl.ds(wid * BW, BW)], pcv)
    pltpu.sync_copy(par_n.at[pl.ds(wid * BW * PN, BW * PN)], pnv)

    def copies(blk, buf):
        yield emb_in.at[itv.at[pl.ds(blk * GB, GB)]], vbuf.at[buf]
        yield emb_out.at[icv.at[pl.ds(blk * GB, GB)]], ubuf.at[buf]
        for i in range(NCHB):
            yield (emb_out.at[inv.at[pl.ds(blk * (GB * NEG) + i * 64, 64)]],
                   nbuf.at[buf, pl.ds(i * 64, 64)])

    def start(blk, buf, sem):
        for src, dst in copies(blk, buf):
            pltpu.async_copy(src, dst, sem)

    def drain(blk, buf, sem):
        for src, dst in copies(blk, buf):
            pltpu.make_async_copy(src, dst, sem).wait()

    last = lax.iota(jnp.int32, 16) == 15
    lane = lax.iota(jnp.int32, 16)

    def bcast(vec, j):
        return jnp.take_along_axis(vec, jnp.full((16,), j, jnp.int32), axis=0)

    def compute(blk, buf):
        vb, ub, nb = vbuf.at[buf], ubuf.at[buf], nbuf.at[buf]

        pvv = ptv[pl.ds(blk * GB, 16)]
        pcc = pcv[pl.ds(blk * GB, 16)]

        def bstep(b, _):
            ob = blk * GB + b
            on = ob * PN
            pn0 = pnv[pl.ds(on, 16)]
            pn1 = pnv[pl.ds(on + 8, 16)]

            def cols(par_vec, j):
                return bcast(par_vec, j) * 64 + lane

            vcol = cols(pvv, b)
            vr = [plsc.load_gather(vb.at[b], [vcol + c_ * 16])
                  for c_ in range(EMB // 16)]
            flat = ob * NP1

            def emit(ref_row, col0, f):
                ss = plsc.load_gather(ref_row, [col0]) * vr[0]
                for c_ in range(1, EMB // 16):
                    ss = ss + plsc.load_gather(ref_row,
                                               [col0 + c_ * 16]) * vr[c_]
                cs = plsc.cumsum(ss)
                fv = jnp.full((16,), f, jnp.int32)
                plsc.store_scatter(dots, [fv], cs, mask=last)

            emit(ub.at[b], cols(pcc, b), flat)
            for k in range(NEG):
                pv_k = cols(pn0, k) if k < 8 else cols(pn1, k - 8)
                emit(nb.at[b * NEG + k], pv_k, flat + 1 + k)
            return 0

        lax.fori_loop(0, GB, bstep, 0)

    start(0, 0, sem0)

    def pairstep(p, _):
        blk = 2 * p
        start(blk + 1, 1, sem1)
        drain(blk, 0, sem0)
        compute(blk, 0)

        @pl.when(blk + 2 < NBLK)
        def _():
            start(blk + 2, 0, sem0)

        drain(blk + 1, 1, sem1)
        compute(blk + 1, 1)
        return 0

    lax.fori_loop(0, NBLK // 2, pairstep, 0)

    pltpu.sync_copy(dots, dots_out.at[pl.ds(wid * (BW * NP1), BW * NP1)])


_dots = functools.partial(
    pl.kernel,
    out_type=jax.ShapeDtypeStruct((BATCH * NP1,), jnp.float32),
    mesh=plsc.VectorSubcoreMesh(core_axis_name="c", subcore_axis_name="s",
                                num_cores=NC, num_subcores=NS),
    scratch_types=[
        pltpu.VMEM((BW,), jnp.int32),
        pltpu.VMEM((BW,), jnp.int32),
        pltpu.VMEM((BW * NEG,), jnp.int32),
        pltpu.VMEM((BW,), jnp.int32),
        pltpu.VMEM((BW,), jnp.int32),
        pltpu.VMEM((BW * PN,), jnp.int32),
        pltpu.VMEM((2, GB, 128), jnp.float32),
        pltpu.VMEM((2, GB, 128), jnp.float32),
        pltpu.VMEM((2, GB * NEG, 128), jnp.float32),
        pltpu.VMEM((BW * NP1,), jnp.float32),
        pltpu.SemaphoreType.DMA,
        pltpu.SemaphoreType.DMA,
    ],
    compiler_params=pltpu.CompilerParams(needs_layout_passes=False),
)(_dots_body)


def _log_sigmoid(x):
    return jnp.minimum(x, 0.0) - jnp.log1p(jnp.exp(-jnp.abs(x)))


def _loss_body(d_ref, out_ref):
    x = d_ref[...]
    r = lax.broadcasted_iota(jnp.int32, (DROWS, DCOLS), 0)
    col = lax.broadcasted_iota(jnp.int32, (DROWS, DCOLS), 1)
    flat = r * DCOLS + col
    y = jnp.where(flat % NP1 == 0, x, -x)
    ls = _log_sigmoid(y)
    out_ref[0, 0] = -jnp.sum(ls) / BATCH


_loss = pl.pallas_call(
    _loss_body,
    in_specs=[pl.BlockSpec((DROWS, DCOLS), lambda: (0, 0))],
    out_specs=pl.BlockSpec(memory_space=pltpu.SMEM),
    out_shape=jax.ShapeDtypeStruct((1, 1), jnp.float32),
)


def kernel(target_input, context, neg, input_emb, output_emb):
    ht = target_input.astype(jnp.int32)
    hc = context.astype(jnp.int32)
    hn = neg.astype(jnp.int32)
    idx_t = ht >> 1
    idx_c = hc >> 1
    idx_n = (hn >> 1).reshape(BATCH * NEG)
    par_t = ht & 1
    par_c = hc & 1
    par_n = jnp.pad(hn & 1, ((0, 0), (0, PN - NEG))).reshape(BATCH * PN)
    def pack(t):
        return jnp.concatenate([t[0::2], t[1::2]], axis=1)

    dots = _dots(pack(input_emb), pack(output_emb),
                 idx_t, idx_c, idx_n, par_t, par_c, par_n)
    return _loss(dots.reshape(DROWS, DCOLS))[0, 0]


# speedup vs baseline: 13.7893x; 13.7893x over previous
import functools

import jax
import jax.numpy as jnp
from jax import lax
from jax.experimental import pallas as pl
from jax.experimental.pallas import tpu as pltpu
from jax.experimental.pallas import tpu_sc as plsc

VOCAB = 1000000
EMB = 64
BATCH = 16384
NEG = 20
NP1 = NEG + 1

NC, NS = 2, 16
NW = NC * NS

BW = BATCH // NW
GB = 16
NBLK = BW // GB
CH = 128
TROWS = BW // CH
NROWS = BW * NEG // CH
NCHB = GB * NEG // 64
PN = 24

DCOLS = 128
DROWS = BATCH * NP1 // DCOLS
DW = DROWS // NW


def _dots_body(emb_in, emb_out, idx_t, idx_c, idx_n, par_t, par_c, par_n,
               dots_out,
               itv, icv, inv, ptv, pcv, pnv, vbuf, ubuf, nbuf, dots,
               sem0, sem1):
    c = lax.axis_index("c")
    s = lax.axis_index("s")
    wid = s * NC + c

    pltpu.sync_copy(idx_t.at[pl.ds(wid * BW, BW)], itv)
    pltpu.sync_copy(idx_c.at[pl.ds(wid * BW, BW)], icv)
    pltpu.sync_copy(idx_n.at[pl.ds(wid * BW * NEG, BW * NEG)], inv)
    pltpu.sync_copy(par_t.at[pl.ds(wid * BW, BW)], ptv)
    pltpu.sync_copy(par_c.at[pl.ds(wid * BW, BW)], pcv)
    pltpu.sync_copy(par_n.at[pl.ds(wid * BW * PN, BW * PN)], pnv)

    def copies(blk, buf):
        yield emb_in.at[itv.at[pl.ds(blk * GB, GB)]], vbuf.at[buf]
        yield emb_out.at[icv.at[pl.ds(blk * GB, GB)]], ubuf.at[buf]
        for i in range(NCHB):
            yield (emb_out.at[inv.at[pl.ds(blk * (GB * NEG) + i * 64, 64)]],
                   nbuf.at[buf, pl.ds(i * 64, 64)])

    def start(blk, buf, sem):
        for src, dst in copies(blk, buf):
            pltpu.async_copy(src, dst, sem)

    def drain(blk, buf, sem):
        for src, dst in copies(blk, buf):
            pltpu.make_async_copy(src, dst, sem).wait()

    last = lax.iota(jnp.int32, 16) == 15
    lane = lax.iota(jnp.int32, 16)

    def bcast(vec, j):
        return jnp.take_along_axis(vec, jnp.full((16,), j, jnp.int32), axis=0)

    def compute(blk, buf):
        vb, ub, nb = vbuf.at[buf], ubuf.at[buf], nbuf.at[buf]

        pvv = ptv[pl.ds(blk * GB, 16)]
        pcc = pcv[pl.ds(blk * GB, 16)]

        def bstep(b, _):
            ob = blk * GB + b
            on = ob * PN
            pn0 = pnv[pl.ds(on, 16)]
            pn1 = pnv[pl.ds(on + 8, 16)]

            def cols(par_vec, j):
                return bcast(par_vec, j) * 64 + lane

            vcol = cols(pvv, b)
            vr = [plsc.load_gather(vb.at[b], [vcol + c_ * 16])
                  for c_ in range(EMB // 16)]
            flat = ob * NP1

            def emit(ref_row, col0, f):
                ss = plsc.load_gather(ref_row, [col0]) * vr[0]
                for c_ in range(1, EMB // 16):
                    ss = ss + plsc.load_gather(ref_row,
                                               [col0 + c_ * 16]) * vr[c_]
                cs = plsc.cumsum(ss)
                fv = jnp.full((16,), f, jnp.int32)
                plsc.store_scatter(dots, [fv], cs, mask=last)

            emit(ub.at[b], cols(pcc, b), flat)
            for k in range(NEG):
                pv_k = cols(pn0, k) if k < 8 else cols(pn1, k - 8)
                emit(nb.at[b * NEG + k], pv_k, flat + 1 + k)
            return 0

        lax.fori_loop(0, GB, bstep, 0)

    start(0, 0, sem0)

    def pairstep(p, _):
        blk = 2 * p
        start(blk + 1, 1, sem1)
        drain(blk, 0, sem0)
        compute(blk, 0)

        @pl.when(blk + 2 < NBLK)
        def _():
            start(blk + 2, 0, sem0)

        drain(blk + 1, 1, sem1)
        compute(blk + 1, 1)
        return 0

    lax.fori_loop(0, NBLK // 2, pairstep, 0)

    pltpu.sync_copy(dots, dots_out.at[pl.ds(wid * (BW * NP1), BW * NP1)])


_dots = functools.partial(
    pl.kernel,
    out_type=jax.ShapeDtypeStruct((BATCH * NP1,), jnp.float32),
    mesh=plsc.VectorSubcoreMesh(core_axis_name="c", subcore_axis_name="s",
                                num_cores=NC, num_subcores=NS),
    scratch_types=[
        pltpu.VMEM((BW,), jnp.int32),
        pltpu.VMEM((BW,), jnp.int32),
        pltpu.VMEM((BW * NEG,), jnp.int32),
        pltpu.VMEM((BW,), jnp.int32),
        pltpu.VMEM((BW,), jnp.int32),
        pltpu.VMEM((BW * PN,), jnp.int32),
        pltpu.VMEM((2, GB, 128), jnp.float32),
        pltpu.VMEM((2, GB, 128), jnp.float32),
        pltpu.VMEM((2, GB * NEG, 128), jnp.float32),
        pltpu.VMEM((BW * NP1,), jnp.float32),
        pltpu.SemaphoreType.DMA,
        pltpu.SemaphoreType.DMA,
    ],
    compiler_params=pltpu.CompilerParams(needs_layout_passes=False),
)(_dots_body)


def _log_sigmoid(x):
    return jnp.minimum(x, 0.0) - jnp.log1p(jnp.exp(-jnp.abs(x)))


def _loss_body(d_ref, out_ref):
    x = d_ref[...]
    r = lax.broadcasted_iota(jnp.int32, (DROWS, DCOLS), 0)
    col = lax.broadcasted_iota(jnp.int32, (DROWS, DCOLS), 1)
    flat = r * DCOLS + col
    y = jnp.where(flat % NP1 == 0, x, -x)
    ls = _log_sigmoid(y)
    out_ref[0, 0] = -jnp.sum(ls) / BATCH


_loss = pl.pallas_call(
    _loss_body,
    in_specs=[pl.BlockSpec((DROWS, DCOLS), lambda: (0, 0))],
    out_specs=pl.BlockSpec(memory_space=pltpu.SMEM),
    out_shape=jax.ShapeDtypeStruct((1, 1), jnp.float32),
)


def kernel(target_input, context, neg, input_emb, output_emb):
    ht = target_input.astype(jnp.int32)
    hc = context.astype(jnp.int32)
    hn = neg.astype(jnp.int32)
    idx_t = ht >> 1
    idx_c = hc >> 1
    idx_n = (hn >> 1).reshape(BATCH * NEG)
    par_t = ht & 1
    par_c = hc & 1
    par_n = jnp.pad(hn & 1, ((0, 0), (0, PN - NEG))).reshape(BATCH * PN)
    dots = _dots(input_emb.reshape(VOCAB // 2, 128),
                 output_emb.reshape(VOCAB // 2, 128),
                 idx_t, idx_c, idx_n, par_t, par_c, par_n)
    return _loss(dots.reshape(DROWS, DCOLS))[0, 0]
